# Initial kernel scaffold; baseline (speedup 1.0000x reference)
import jax
import jax.numpy as jnp
from jax.experimental import pallas as pl


def kernel(x, edge_index, edge_weights, W, gn_gamma, gn_beta, gn_alpha, W1, b1, W2, b2):
    raise NotImplementedError("write your pallas kernel here")


# speedup vs baseline: 6.3744x; 6.3744x over previous
import jax
import jax.numpy as jnp
from jax import lax
from jax.experimental import pallas as pl
from jax.experimental.pallas import tpu as pltpu
from jax.experimental.pallas import tpu_sc as plsc

N_NODES = 10000
N_EDGES = 320000
D = 128
OUT_D = 16
EPS = 1e-05

NC = 2
NS = 16
NW = NC * NS
RB = 128
NR_TOT = 2504
E_PAD = NR_TOT * RB
N_PAD_EDGES = E_PAD - N_EDGES

NR_HI = 80
NR_LO = 72
NR_SPLIT = 25

ZROWS = 640


def _mesh():
  return plsc.VectorSubcoreMesh(
      core_axis_name="c", subcore_axis_name="s", num_cores=NC, num_subcores=NS)


def _chunk(w):
  rbase = jnp.where(w < NR_SPLIT, NR_HI * w,
                    NR_HI * NR_SPLIT + NR_LO * (w - NR_SPLIT))
  nr = jnp.where(w < NR_SPLIT, NR_HI, NR_LO)
  return pl.multiple_of(rbase, 8), nr


def _stage_chunk(w, rbase, edge3, ew2, src_i, dst_i, ew_v):
  @pl.when(w < NR_SPLIT)
  def _():
    pltpu.sync_copy(edge3.at[0, pl.ds(rbase, NR_HI)], src_i)
    pltpu.sync_copy(edge3.at[1, pl.ds(rbase, NR_HI)], dst_i)
    if ew_v is not None:
      pltpu.sync_copy(ew2.at[pl.ds(rbase, NR_HI)], ew_v)

  @pl.when(w >= NR_SPLIT)
  def _():
    pltpu.sync_copy(edge3.at[0, pl.ds(rbase, NR_LO)], src_i.at[pl.ds(0, NR_LO)])
    pltpu.sync_copy(edge3.at[1, pl.ds(rbase, NR_LO)], dst_i.at[pl.ds(0, NR_LO)])
    if ew_v is not None:
      pltpu.sync_copy(ew2.at[pl.ds(rbase, NR_LO)], ew_v.at[pl.ds(0, NR_LO)])


def _deg_body(edge3, dpart, od_sh, id_sh, src_i, dst_i, ones_v, zbuf):
  c = lax.axis_index("c")
  s = lax.axis_index("s")
  w = c * NS + s

  z16 = jnp.zeros((16,), jnp.float32)
  o16 = jnp.ones((16,), jnp.float32)

  def _zb(i, _):
    zbuf[pl.ds(i * 16, 16)] = z16
    return 0

  lax.fori_loop(0, N_NODES // 16, _zb, 0)
  for k in range(RB // 16):
    ones_v[pl.ds(k * 16, 16)] = o16

  @pl.when(s == 0)
  def _():
    pltpu.sync_copy(zbuf, od_sh)

  @pl.when(s == 1)
  def _():
    pltpu.sync_copy(zbuf, id_sh)

  rbase, nr = _chunk(w)
  _stage_chunk(w, rbase, edge3, None, src_i, dst_i, None)

  plsc.subcore_barrier()

  def _round(r, _):
    pltpu.sync_copy(ones_v, od_sh.at[src_i.at[r]], add=True)
    pltpu.sync_copy(ones_v, id_sh.at[dst_i.at[r]], add=True)
    return 0

  lax.fori_loop(0, nr, _round, 0)
  plsc.subcore_barrier()

  @pl.when(s == 0)
  def _():
    off = pl.multiple_of(2 * c * N_NODES, 8)
    pltpu.sync_copy(od_sh, zbuf)
    pltpu.sync_copy(zbuf, dpart.at[pl.ds(off, N_NODES)])

  @pl.when(s == 1)
  def _():
    off = pl.multiple_of((2 * c + 1) * N_NODES, 8)
    pltpu.sync_copy(id_sh, zbuf)
    pltpu.sync_copy(zbuf, dpart.at[pl.ds(off, N_NODES)])


def _deg_call(edge3):
  return pl.kernel(
      _deg_body,
      out_type=jax.ShapeDtypeStruct((4 * N_NODES,), jnp.float32),
      mesh=_mesh(),
      scratch_types=[
          pltpu.VMEM_SHARED((N_NODES,), jnp.float32),
          pltpu.VMEM_SHARED((N_NODES,), jnp.float32),
          pltpu.VMEM((NR_HI, RB), jnp.int32),
          pltpu.VMEM((NR_HI, RB), jnp.int32),
          pltpu.VMEM((RB,), jnp.float32),
          pltpu.VMEM((N_NODES,), jnp.float32),
      ],
  )(edge3)


PBLK = 1000


def _prep_body(dp_ref, x_ref, h_ref, di_ref):
  i = pl.program_id(0)
  dp = dp_ref[...]
  rows0 = lax.broadcasted_iota(jnp.int32, (PBLK, 1), 0) == 0
  corr = jnp.where(rows0 & (i == 0), jnp.float32(N_PAD_EDGES), 0.0)
  od = jnp.maximum(dp[0] + dp[2] - corr, 1.0)
  idg = jnp.maximum(dp[1] + dp[3] - corr, 1.0)
  h_ref[...] = x_ref[...] * lax.rsqrt(od)
  di_ref[...] = lax.rsqrt(idg)


def _prep_call(deg_part, x):
  dp3 = deg_part.reshape(4, N_NODES, 1)
  return pl.pallas_call(
      _prep_body,
      grid=(N_NODES // PBLK,),
      in_specs=[
          pl.BlockSpec((4, PBLK, 1), lambda i: (0, i, 0)),
          pl.BlockSpec((PBLK, D), lambda i: (i, 0)),
      ],
      out_specs=[
          pl.BlockSpec((PBLK, D), lambda i: (i, 0)),
          pl.BlockSpec((PBLK, 1), lambda i: (i, 0)),
      ],
      out_shape=[
          jax.ShapeDtypeStruct((N_NODES, D), jnp.float32),
          jax.ShapeDtypeStruct((N_NODES, 1), jnp.float32),
      ],
  )(dp3, x)


_LANE_IDX = None


def _agg_body(edge3, ew2, h, dinvi, apart,
              src_i, dst_i, ew_v, div_v, rows, agg_sh):
  c = lax.axis_index("c")
  s = lax.axis_index("s")
  w = c * NS + s

  z16 = jnp.zeros((16,), jnp.float32)

  def _zr(e, _):
    for ct in range(D // 16):
      rows[e, pl.ds(ct * 16, 16)] = z16
    return 0

  lax.fori_loop(0, RB, _zr, 0)

  for k5 in range(ZROWS // RB):
    start = pl.multiple_of(s * ZROWS + k5 * RB, 8)

    @pl.when(start + RB <= N_NODES)
    def _():
      pltpu.sync_copy(rows, agg_sh.at[pl.ds(start, RB)])

  @pl.when(s == NS - 1)
  def _():
    pltpu.sync_copy(rows.at[pl.ds(0, 16)], agg_sh.at[pl.ds(N_NODES - 16, 16)])

  rbase, nr = _chunk(w)
  _stage_chunk(w, rbase, edge3, ew2, src_i, dst_i, ew_v)

  plsc.subcore_barrier()

  def _round(r, _):
    pltpu.sync_copy(h.at[src_i.at[r]], rows)
    pltpu.sync_copy(dinvi.at[dst_i.at[r]], div_v)
    for grp in range(RB // 16):
      sv16 = ew_v[r, pl.ds(grp * 16, 16)] * div_v[pl.ds(grp * 16, 16)]
      for lane in range(16):
        e = grp * 16 + lane
        sp = sv16.at[jnp.full((16,), lane, jnp.int32)].get(
            mode="promise_in_bounds")
        for ct in range(D // 16):
          rows[e, pl.ds(ct * 16, 16)] = rows[e, pl.ds(ct * 16, 16)] * sp
    pltpu.sync_copy(rows, agg_sh.at[dst_i.at[r]], add=True)
    return 0

  lax.fori_loop(0, nr, _round, 0)
  plsc.subcore_barrier()

  for k5 in range(ZROWS // RB):
    start = pl.multiple_of(s * ZROWS + k5 * RB, 8)

    @pl.when(start + RB <= N_NODES)
    def _():
      pltpu.sync_copy(agg_sh.at[pl.ds(start, RB)], rows)
      pltpu.sync_copy(rows, apart.at[c, pl.ds(start, RB)])

  @pl.when(s == NS - 1)
  def _():
    pltpu.sync_copy(agg_sh.at[pl.ds(N_NODES - 16, 16)], rows.at[pl.ds(0, 16)])
    pltpu.sync_copy(rows.at[pl.ds(0, 16)], apart.at[c, pl.ds(N_NODES - 16, 16)])


def _agg_call(edge3, ew2, h, dinvi):
  return pl.kernel(
      _agg_body,
      out_type=jax.ShapeDtypeStruct((NC, N_NODES, D), jnp.float32),
      mesh=_mesh(),
      scratch_types=[
          pltpu.VMEM((NR_HI, RB), jnp.int32),
          pltpu.VMEM((NR_HI, RB), jnp.int32),
          pltpu.VMEM((NR_HI, RB), jnp.float32),
          pltpu.VMEM((RB,), jnp.float32),
          pltpu.VMEM((RB, D), jnp.float32),
          pltpu.VMEM_SHARED((N_NODES, D), jnp.float32),
      ],
  )(edge3, ew2, h, dinvi)


RBLK = 1000


def _head_body(a2_ref, w_ref, gng_ref, gnb_ref, gna_ref,
               w1_ref, b1_ref, w2_ref, b2_ref, out_ref,
               s1, s2, mx, mn):
  i = pl.program_id(0)
  a = a2_ref[0] + a2_ref[1]
  h = jnp.dot(a, w_ref[...], preferred_element_type=jnp.float32)
  h = jnp.where(h > 0, h, 0.01 * h)
  bs1 = jnp.sum(h, axis=0, keepdims=True)
  bs2 = jnp.sum(h * h, axis=0, keepdims=True)
  bmx = jnp.max(h, axis=0, keepdims=True)
  bmn = jnp.min(h, axis=0, keepdims=True)

  @pl.when(i == 0)
  def _():
    s1[...] = bs1
    s2[...] = bs2
    mx[...] = bmx
    mn[...] = bmn

  @pl.when(i > 0)
  def _():
    s1[...] += bs1
    s2[...] += bs2
    mx[...] = jnp.maximum(mx[...], bmx)
    mn[...] = jnp.minimum(mn[...], bmn)

  @pl.when(i == pl.num_programs(0) - 1)
  def _():
    nf = jnp.float32(N_NODES)
    mu = s1[...] / nf
    al = gna_ref[...]
    var = s2[...] / nf + mu * mu * (al * al - 2.0 * al)
    sig = lax.rsqrt(var + EPS)
    gam = gng_ref[...]
    bet = gnb_ref[...]
    gs = gam * sig
    g_mean = gs * (mu - al * mu) + bet
    hc_mx = mx[...] - al * mu
    hc_mn = mn[...] - al * mu
    g_max = jnp.where(gam >= 0, gs * hc_mx, gs * hc_mn) + bet
    g_min = jnp.where(gam >= 0, gs * hc_mn, gs * hc_mx) + bet
    g = jnp.concatenate([g_mean, g_max, g_min], axis=1)
    hid = jnp.maximum(
        jnp.dot(g, w1_ref[...], preferred_element_type=jnp.float32)
        + b1_ref[...], 0.0)
    out_ref[...] = (jnp.dot(hid, w2_ref[...], preferred_element_type=jnp.float32)
                    + b2_ref[...])


def _head_call(apart, W, gng, gnb, gna, W1, b1, W2, b2):
  nsteps = N_NODES // RBLK
  full = lambda shape: pl.BlockSpec(shape, lambda i: tuple(0 for _ in shape))
  return pl.pallas_call(
      _head_body,
      grid=(nsteps,),
      in_specs=[
          pl.BlockSpec((NC, RBLK, D), lambda i: (0, i, 0)),
          full((D, D)),
          full((1, D)),
          full((1, D)),
          full((1, D)),
          full((3 * D, 256)),
          full((1, 256)),
          full((256, OUT_D)),
          full((1, OUT_D)),
      ],
      out_specs=pl.BlockSpec((1, OUT_D), lambda i: (0, 0)),
      out_shape=jax.ShapeDtypeStruct((1, OUT_D), jnp.float32),
      scratch_shapes=[pltpu.VMEM((1, D), jnp.float32) for _ in range(4)],
  )(apart, W, gng, gnb, gna, W1, b1, W2, b2)


def kernel(x, edge_index, edge_weights, W, gn_gamma, gn_beta, gn_alpha,
           W1, b1, W2, b2):
  edge3 = jnp.pad(edge_index, ((0, 0), (0, N_PAD_EDGES))).reshape(2, NR_TOT, RB)
  ew2 = jnp.pad(edge_weights, (0, N_PAD_EDGES)).reshape(NR_TOT, RB)
  deg_part = _deg_call(edge3)
  h, dinvi2 = _prep_call(deg_part, x)
  apart = _agg_call(edge3, ew2, h, dinvi2.reshape(N_NODES))
  return _head_call(
      apart, W,
      gn_gamma.reshape(1, D), gn_beta.reshape(1, D), gn_alpha.reshape(1, D),
      W1, b1.reshape(1, -1), W2, b2.reshape(1, -1))


# speedup vs baseline: 9.7818x; 1.5345x over previous
import jax
import jax.numpy as jnp
from jax import lax
from jax.experimental import pallas as pl
from jax.experimental.pallas import tpu as pltpu
from jax.experimental.pallas import tpu_sc as plsc

N_NODES = 10000
N_EDGES = 320000
D = 128
OUT_D = 16
EPS = 1e-05

NC = 2
NS = 16
NW = NC * NS
RB = 128
NR_TOT = 2504
E_PAD = NR_TOT * RB
N_PAD_EDGES = E_PAD - N_EDGES

NR_HI = 80
NR_LO = 72
NR_SPLIT = 25

RB3 = 64

ZROWS = 640


def _mesh():
  return plsc.VectorSubcoreMesh(
      core_axis_name="c", subcore_axis_name="s", num_cores=NC, num_subcores=NS)


def _chunk(w):
  rbase = jnp.where(w < NR_SPLIT, NR_HI * w,
                    NR_HI * NR_SPLIT + NR_LO * (w - NR_SPLIT))
  nr = jnp.where(w < NR_SPLIT, NR_HI, NR_LO)
  return pl.multiple_of(rbase, 8), nr


def _stage_chunk(w, rbase, edge3, ew2, src_i, dst_i, ew_v):
  @pl.when(w < NR_SPLIT)
  def _():
    pltpu.sync_copy(edge3.at[0, pl.ds(rbase, NR_HI)], src_i)
    pltpu.sync_copy(edge3.at[1, pl.ds(rbase, NR_HI)], dst_i)
    if ew_v is not None:
      pltpu.sync_copy(ew2.at[pl.ds(rbase, NR_HI)], ew_v)

  @pl.when(w >= NR_SPLIT)
  def _():
    pltpu.sync_copy(edge3.at[0, pl.ds(rbase, NR_LO)], src_i.at[pl.ds(0, NR_LO)])
    pltpu.sync_copy(edge3.at[1, pl.ds(rbase, NR_LO)], dst_i.at[pl.ds(0, NR_LO)])
    if ew_v is not None:
      pltpu.sync_copy(ew2.at[pl.ds(rbase, NR_LO)], ew_v.at[pl.ds(0, NR_LO)])


def _deg_body(edge3, dpart, od_sh, id_sh, src_i, dst_i, ones_v, zbuf):
  c = lax.axis_index("c")
  s = lax.axis_index("s")
  w = c * NS + s

  z16 = jnp.zeros((16,), jnp.float32)
  o16 = jnp.ones((16,), jnp.float32)

  def _zb(i, _):
    zbuf[pl.ds(i * 16, 16)] = z16
    return 0

  lax.fori_loop(0, N_NODES // 16, _zb, 0)
  for k in range(RB // 16):
    ones_v[pl.ds(k * 16, 16)] = o16

  @pl.when(s == 0)
  def _():
    pltpu.sync_copy(zbuf, od_sh)

  @pl.when(s == 1)
  def _():
    pltpu.sync_copy(zbuf, id_sh)

  rbase, nr = _chunk(w)
  _stage_chunk(w, rbase, edge3, None, src_i, dst_i, None)

  plsc.subcore_barrier()

  def _round(r, _):
    pltpu.sync_copy(ones_v, od_sh.at[src_i.at[r]], add=True)
    pltpu.sync_copy(ones_v, id_sh.at[dst_i.at[r]], add=True)
    return 0

  lax.fori_loop(0, nr, _round, 0)
  plsc.subcore_barrier()

  @pl.when(s == 0)
  def _():
    off = pl.multiple_of(2 * c * N_NODES, 8)
    pltpu.sync_copy(od_sh, zbuf)
    pltpu.sync_copy(zbuf, dpart.at[pl.ds(off, N_NODES)])

  @pl.when(s == 1)
  def _():
    off = pl.multiple_of((2 * c + 1) * N_NODES, 8)
    pltpu.sync_copy(id_sh, zbuf)
    pltpu.sync_copy(zbuf, dpart.at[pl.ds(off, N_NODES)])


def _deg_call(edge3):
  return pl.kernel(
      _deg_body,
      out_type=jax.ShapeDtypeStruct((4 * N_NODES,), jnp.float32),
      mesh=_mesh(),
      scratch_types=[
          pltpu.VMEM_SHARED((N_NODES,), jnp.float32),
          pltpu.VMEM_SHARED((N_NODES,), jnp.float32),
          pltpu.VMEM((NR_HI, RB), jnp.int32),
          pltpu.VMEM((NR_HI, RB), jnp.int32),
          pltpu.VMEM((RB,), jnp.float32),
          pltpu.VMEM((N_NODES,), jnp.float32),
      ],
  )(edge3)


PBLK = 1000


def _prep_body(dp_ref, x_ref, h_ref, di_ref):
  i = pl.program_id(0)
  dp = dp_ref[...]
  rows0 = lax.broadcasted_iota(jnp.int32, (PBLK, 1), 0) == 0
  corr = jnp.where(rows0 & (i == 0), jnp.float32(N_PAD_EDGES), 0.0)
  od = jnp.maximum(dp[0] + dp[2] - corr, 1.0)
  idg = jnp.maximum(dp[1] + dp[3] - corr, 1.0)
  h_ref[...] = x_ref[...] * lax.rsqrt(od)
  di_ref[...] = lax.rsqrt(idg)


def _prep_call(deg_part, x):
  dp3 = deg_part.reshape(4, N_NODES, 1)
  return pl.pallas_call(
      _prep_body,
      grid=(N_NODES // PBLK,),
      in_specs=[
          pl.BlockSpec((4, PBLK, 1), lambda i: (0, i, 0)),
          pl.BlockSpec((PBLK, D), lambda i: (i, 0)),
      ],
      out_specs=[
          pl.BlockSpec((PBLK, D), lambda i: (i, 0)),
          pl.BlockSpec((PBLK, 1), lambda i: (i, 0)),
      ],
      out_shape=[
          jax.ShapeDtypeStruct((N_NODES, D), jnp.float32),
          jax.ShapeDtypeStruct((N_NODES, 1), jnp.float32),
      ],
  )(dp3, x)


def _agg_body(edge3, ew2, h, apart,
              src_i, dst_i, ew_v, gbig, idx64, agg_sh, sem_g):
  c = lax.axis_index("c")
  s = lax.axis_index("s")
  w = c * NS + s
  r0 = gbig.at[pl.ds(0, RB)]

  z16 = jnp.zeros((16,), jnp.float32)

  def _zr(e, _):
    for ct in range(D // 16):
      r0[e, pl.ds(ct * 16, 16)] = z16
    return 0

  lax.fori_loop(0, RB, _zr, 0)

  for k5 in range(ZROWS // RB):
    start = pl.multiple_of(s * ZROWS + k5 * RB, 8)

    @pl.when(start + RB <= N_NODES)
    def _():
      pltpu.sync_copy(r0, agg_sh.at[pl.ds(start, RB)])

  @pl.when(s == NS - 1)
  def _():
    pltpu.sync_copy(r0.at[pl.ds(0, 16)], agg_sh.at[pl.ds(N_NODES - 16, 16)])

  rbase, nrow = _chunk(w)
  _stage_chunk(w, rbase, edge3, ew2, src_i, dst_i, ew_v)

  plsc.subcore_barrier()

  gbufs = [gbig.at[pl.ds(0, RB3)], gbig.at[pl.ds(RB3, RB3)]]

  def _srcidx(row, half):
    return src_i.at[row, pl.ds(half * RB3, RB3)]

  def _scale(gb, row, half):
    for grp in range(RB3 // 16):
      sv16 = ew_v[row, pl.ds(half * RB3 + grp * 16, 16)]
      for lane in range(16):
        e = grp * 16 + lane
        sp = sv16.at[jnp.full((16,), lane, jnp.int32)].get(
            mode="promise_in_bounds")
        for ct in range(D // 16):
          gb[e, pl.ds(ct * 16, 16)] = gb[e, pl.ds(ct * 16, 16)] * sp

  def _step(rr, _):
    row = rr - 1
    for half in range(2):
      gb = gbufs[half]

      @pl.when(row >= 0)
      def _():
        pltpu.make_async_copy(h.at[_srcidx(row, half)], gb,
                              sem_g.at[half]).wait()
        _scale(gb, row, half)
        for k in range(RB3 // 16):
          idx64[pl.ds(k * 16, 16)] = dst_i[row, pl.ds(half * RB3 + k * 16, 16)]
        pltpu.sync_copy(gb, agg_sh.at[idx64], add=True)

      @pl.when(rr < nrow)
      def _():
        pltpu.async_copy(h.at[_srcidx(rr, half)], gb, sem_g.at[half])
    return 0

  lax.fori_loop(0, nrow + 1, _step, 0)

  plsc.subcore_barrier()

  for k5 in range(ZROWS // RB):
    start = pl.multiple_of(s * ZROWS + k5 * RB, 8)

    @pl.when(start + RB <= N_NODES)
    def _():
      pltpu.sync_copy(agg_sh.at[pl.ds(start, RB)], r0)
      pltpu.sync_copy(r0, apart.at[c, pl.ds(start, RB)])

  @pl.when(s == NS - 1)
  def _():
    pltpu.sync_copy(agg_sh.at[pl.ds(N_NODES - 16, 16)], r0.at[pl.ds(0, 16)])
    pltpu.sync_copy(r0.at[pl.ds(0, 16)], apart.at[c, pl.ds(N_NODES - 16, 16)])


def _agg_call(edge3, ew2, h):
  return pl.kernel(
      _agg_body,
      out_type=jax.ShapeDtypeStruct((NC, N_NODES, D), jnp.float32),
      mesh=_mesh(),
      scratch_types=(
          [
              pltpu.VMEM((NR_HI, RB), jnp.int32),
              pltpu.VMEM((NR_HI, RB), jnp.int32),
              pltpu.VMEM((NR_HI, RB), jnp.float32),
          ]
          + [pltpu.VMEM((2 * RB3, D), jnp.float32)]
          + [pltpu.VMEM((RB3,), jnp.int32)]
          + [pltpu.VMEM_SHARED((N_NODES, D), jnp.float32)]
          + [pltpu.SemaphoreType.DMA((2,))]
      ),
  )(edge3, ew2, h)


RBLK = 1000


def _head_body(a2_ref, di_ref, w_ref, gng_ref, gnb_ref, gna_ref,
               w1_ref, b1_ref, w2_ref, b2_ref, out_ref,
               s1, s2, mx, mn):
  i = pl.program_id(0)
  a = (a2_ref[0] + a2_ref[1]) * di_ref[...]
  h = jnp.dot(a, w_ref[...], preferred_element_type=jnp.float32)
  h = jnp.where(h > 0, h, 0.01 * h)
  bs1 = jnp.sum(h, axis=0, keepdims=True)
  bs2 = jnp.sum(h * h, axis=0, keepdims=True)
  bmx = jnp.max(h, axis=0, keepdims=True)
  bmn = jnp.min(h, axis=0, keepdims=True)

  @pl.when(i == 0)
  def _():
    s1[...] = bs1
    s2[...] = bs2
    mx[...] = bmx
    mn[...] = bmn

  @pl.when(i > 0)
  def _():
    s1[...] += bs1
    s2[...] += bs2
    mx[...] = jnp.maximum(mx[...], bmx)
    mn[...] = jnp.minimum(mn[...], bmn)

  @pl.when(i == pl.num_programs(0) - 1)
  def _():
    nf = jnp.float32(N_NODES)
    mu = s1[...] / nf
    al = gna_ref[...]
    var = s2[...] / nf + mu * mu * (al * al - 2.0 * al)
    sig = lax.rsqrt(var + EPS)
    gam = gng_ref[...]
    bet = gnb_ref[...]
    gs = gam * sig
    g_mean = gs * (mu - al * mu) + bet
    hc_mx = mx[...] - al * mu
    hc_mn = mn[...] - al * mu
    g_max = jnp.where(gam >= 0, gs * hc_mx, gs * hc_mn) + bet
    g_min = jnp.where(gam >= 0, gs * hc_mn, gs * hc_mx) + bet
    g = jnp.concatenate([g_mean, g_max, g_min], axis=1)
    hid = jnp.maximum(
        jnp.dot(g, w1_ref[...], preferred_element_type=jnp.float32)
        + b1_ref[...], 0.0)
    out_ref[...] = (jnp.dot(hid, w2_ref[...], preferred_element_type=jnp.float32)
                    + b2_ref[...])


def _head_call(apart, dinvi2, W, gng, gnb, gna, W1, b1, W2, b2):
  nsteps = N_NODES // RBLK
  full = lambda shape: pl.BlockSpec(shape, lambda i: tuple(0 for _ in shape))
  return pl.pallas_call(
      _head_body,
      grid=(nsteps,),
      in_specs=[
          pl.BlockSpec((NC, RBLK, D), lambda i: (0, i, 0)),
          pl.BlockSpec((RBLK, 1), lambda i: (i, 0)),
          full((D, D)),
          full((1, D)),
          full((1, D)),
          full((1, D)),
          full((3 * D, 256)),
          full((1, 256)),
          full((256, OUT_D)),
          full((1, OUT_D)),
      ],
      out_specs=pl.BlockSpec((1, OUT_D), lambda i: (0, 0)),
      out_shape=jax.ShapeDtypeStruct((1, OUT_D), jnp.float32),
      scratch_shapes=[pltpu.VMEM((1, D), jnp.float32) for _ in range(4)],
  )(apart, dinvi2, W, gng, gnb, gna, W1, b1, W2, b2)


def kernel(x, edge_index, edge_weights, W, gn_gamma, gn_beta, gn_alpha,
           W1, b1, W2, b2):
  edge3 = jnp.pad(edge_index, ((0, 0), (0, N_PAD_EDGES))).reshape(2, NR_TOT, RB)
  ew2 = jnp.pad(edge_weights, (0, N_PAD_EDGES)).reshape(NR_TOT, RB)
  deg_part = _deg_call(edge3)
  h, dinvi2 = _prep_call(deg_part, x)
  apart = _agg_call(edge3, ew2, h)
  return _head_call(
      apart, dinvi2, W,
      gn_gamma.reshape(1, D), gn_beta.reshape(1, D), gn_alpha.reshape(1, D),
      W1, b1.reshape(1, -1), W2, b2.reshape(1, -1))


# speedup vs baseline: 10.4762x; 1.0710x over previous
import jax
import jax.numpy as jnp
from jax import lax
from jax.experimental import pallas as pl
from jax.experimental.pallas import tpu as pltpu
from jax.experimental.pallas import tpu_sc as plsc

N_NODES = 10000
N_EDGES = 320000
D = 128
OUT_D = 16
EPS = 1e-05

NC = 2
NS = 16
NW = NC * NS
RB = 128
NR_TOT = 2504
E_PAD = NR_TOT * RB
N_PAD_EDGES = E_PAD - N_EDGES

NR_HI = 80
NR_LO = 72
NR_SPLIT = 25

RB3 = 64

ZROWS = 640


def _mesh():
  return plsc.VectorSubcoreMesh(
      core_axis_name="c", subcore_axis_name="s", num_cores=NC, num_subcores=NS)


def _chunk(w):
  rbase = jnp.where(w < NR_SPLIT, NR_HI * w,
                    NR_HI * NR_SPLIT + NR_LO * (w - NR_SPLIT))
  nr = jnp.where(w < NR_SPLIT, NR_HI, NR_LO)
  return pl.multiple_of(rbase, 8), nr


def _stage_chunk(w, rbase, edge3, ew2, src_i, dst_i, ew_v):
  @pl.when(w < NR_SPLIT)
  def _():
    pltpu.sync_copy(edge3.at[0, pl.ds(rbase, NR_HI)], src_i)
    pltpu.sync_copy(edge3.at[1, pl.ds(rbase, NR_HI)], dst_i)
    if ew_v is not None:
      pltpu.sync_copy(ew2.at[pl.ds(rbase, NR_HI)], ew_v)

  @pl.when(w >= NR_SPLIT)
  def _():
    pltpu.sync_copy(edge3.at[0, pl.ds(rbase, NR_LO)], src_i.at[pl.ds(0, NR_LO)])
    pltpu.sync_copy(edge3.at[1, pl.ds(rbase, NR_LO)], dst_i.at[pl.ds(0, NR_LO)])
    if ew_v is not None:
      pltpu.sync_copy(ew2.at[pl.ds(rbase, NR_LO)], ew_v.at[pl.ds(0, NR_LO)])


def _deg_body(edge3, dpart, od_sh, id_sh, src_i, dst_i, ones_v, zbuf):
  c = lax.axis_index("c")
  s = lax.axis_index("s")
  w = c * NS + s

  z16 = jnp.zeros((16,), jnp.float32)
  o16 = jnp.ones((16,), jnp.float32)

  def _zb(i, _):
    zbuf[pl.ds(i * 16, 16)] = z16
    return 0

  lax.fori_loop(0, N_NODES // 16, _zb, 0)
  for k in range(RB // 16):
    ones_v[pl.ds(k * 16, 16)] = o16

  @pl.when(s == 0)
  def _():
    pltpu.sync_copy(zbuf, od_sh)

  @pl.when(s == 1)
  def _():
    pltpu.sync_copy(zbuf, id_sh)

  rbase, nr = _chunk(w)
  _stage_chunk(w, rbase, edge3, None, src_i, dst_i, None)

  plsc.subcore_barrier()

  def _round(r, _):
    pltpu.sync_copy(ones_v, od_sh.at[src_i.at[r]], add=True)
    pltpu.sync_copy(ones_v, id_sh.at[dst_i.at[r]], add=True)
    return 0

  lax.fori_loop(0, nr, _round, 0)
  plsc.subcore_barrier()

  @pl.when(s == 0)
  def _():
    off = pl.multiple_of(2 * c * N_NODES, 8)
    pltpu.sync_copy(od_sh, zbuf)
    pltpu.sync_copy(zbuf, dpart.at[pl.ds(off, N_NODES)])

  @pl.when(s == 1)
  def _():
    off = pl.multiple_of((2 * c + 1) * N_NODES, 8)
    pltpu.sync_copy(id_sh, zbuf)
    pltpu.sync_copy(zbuf, dpart.at[pl.ds(off, N_NODES)])


def _deg_call(edge3):
  return pl.kernel(
      _deg_body,
      out_type=jax.ShapeDtypeStruct((4 * N_NODES,), jnp.float32),
      mesh=_mesh(),
      scratch_types=[
          pltpu.VMEM_SHARED((N_NODES,), jnp.float32),
          pltpu.VMEM_SHARED((N_NODES,), jnp.float32),
          pltpu.VMEM((NR_HI, RB), jnp.int32),
          pltpu.VMEM((NR_HI, RB), jnp.int32),
          pltpu.VMEM((RB,), jnp.float32),
          pltpu.VMEM((N_NODES,), jnp.float32),
      ],
  )(edge3)


PBLK = 1000


def _prep_body(dp_ref, x_ref, h_ref, di_ref):
  i = pl.program_id(0)
  dp = dp_ref[...]
  rows0 = lax.broadcasted_iota(jnp.int32, (PBLK, 1), 0) == 0
  corr = jnp.where(rows0 & (i == 0), jnp.float32(N_PAD_EDGES), 0.0)
  od = jnp.maximum(dp[0] + dp[2] - corr, 1.0)
  idg = jnp.maximum(dp[1] + dp[3] - corr, 1.0)
  h_ref[...] = x_ref[...] * lax.rsqrt(od)
  di_ref[...] = lax.rsqrt(idg)


def _prep_call(deg_part, x):
  dp3 = deg_part.reshape(4, N_NODES, 1)
  return pl.pallas_call(
      _prep_body,
      grid=(N_NODES // PBLK,),
      in_specs=[
          pl.BlockSpec((4, PBLK, 1), lambda i: (0, i, 0)),
          pl.BlockSpec((PBLK, D), lambda i: (i, 0)),
      ],
      out_specs=[
          pl.BlockSpec((PBLK, D), lambda i: (i, 0)),
          pl.BlockSpec((PBLK, 1), lambda i: (i, 0)),
      ],
      out_shape=[
          jax.ShapeDtypeStruct((N_NODES, D), jnp.float32),
          jax.ShapeDtypeStruct((N_NODES, 1), jnp.float32),
      ],
  )(dp3, x)


def _agg_body(pk2, ew3, h, apart,
              pk_i, gbig, stg0, stg1, ew_roll, gidx0, gidx1, sidx0, sidx1,
              agg_sh, sem_g, sem_e, sem_s0, sem_s1):
  c = lax.axis_index("c")
  s = lax.axis_index("s")
  w = c * NS + s
  r0 = gbig.at[pl.ds(0, RB)]

  z16 = jnp.zeros((16,), jnp.float32)

  def _zr(e, _):
    for ct in range(D // 16):
      r0[e, pl.ds(ct * 16, 16)] = z16
    return 0

  lax.fori_loop(0, RB, _zr, 0)

  for k5 in range(ZROWS // RB):
    start = pl.multiple_of(s * ZROWS + k5 * RB, 8)

    @pl.when(start + RB <= N_NODES)
    def _():
      pltpu.sync_copy(r0, agg_sh.at[pl.ds(start, RB)])

  @pl.when(s == NS - 1)
  def _():
    pltpu.sync_copy(r0.at[pl.ds(0, 16)], agg_sh.at[pl.ds(N_NODES - 16, 16)])

  rbase, nrow = _chunk(w)

  @pl.when(w < NR_SPLIT)
  def _():
    pltpu.sync_copy(pk2.at[pl.ds(rbase, NR_HI)], pk_i)

  @pl.when(w >= NR_SPLIT)
  def _():
    pltpu.sync_copy(pk2.at[pl.ds(rbase, NR_LO)], pk_i.at[pl.ds(0, NR_LO)])

  plsc.subcore_barrier()

  gbufs = [gbig.at[pl.ds(0, RB3)], gbig.at[pl.ds(RB3, RB3)]]
  stgs = [stg0, stg1]
  gidxs = [gidx0, gidx1]
  sidxs = [sidx0, sidx1]
  sem_ss = [sem_s0, sem_s1]

  def _scale(gb, stg, row, half):
    par = row % 2
    for grp in range(RB3 // 16):
      sv16 = ew_roll[par, pl.ds(half * RB3 + grp * 16, 16)]
      for lane in range(16):
        e = grp * 16 + lane
        sp = sv16.at[jnp.full((16,), lane, jnp.int32)].get(
            mode="promise_in_bounds")
        for ct in range(D // 16):
          stg[e, pl.ds(ct * 16, 16)] = gb[e, pl.ds(ct * 16, 16)] * sp

  def _step(rr, _):
    row = rr - 1

    @pl.when(row >= 0)
    def _():
      pltpu.make_async_copy(ew3.at[rbase + row],
                            ew_roll.at[pl.ds(row % 2, 1)],
                            sem_e.at[row % 2]).wait()

    for half in range(2):
      gb = gbufs[half]
      stg = stgs[half]
      sidx = sidxs[half]
      sem_s = sem_ss[half]

      @pl.when(row >= 0)
      def _():
        pltpu.make_async_copy(h.at[gidxs[half]], gb, sem_g.at[half]).wait()

        @pl.when(row >= 1)
        def _():
          pltpu.make_async_copy(stg, agg_sh.at[sidx], sem_s).wait()

        _scale(gb, stg, row, half)
        for k in range(RB3 // 16):
          sidx[pl.ds(k * 16, 16)] = jnp.right_shift(
              pk_i[row, pl.ds(half * RB3 + k * 16, 16)], 16)
        pltpu.async_copy(stg, agg_sh.at[sidx], sem_s, add=True)

      @pl.when(rr < nrow)
      def _():
        for k in range(RB3 // 16):
          gidxs[half][pl.ds(k * 16, 16)] = (
              pk_i[rr, pl.ds(half * RB3 + k * 16, 16)] & 0xFFFF)
        pltpu.async_copy(h.at[gidxs[half]], gb, sem_g.at[half])

    @pl.when(rr < nrow)
    def _():
      pltpu.async_copy(ew3.at[rbase + rr], ew_roll.at[pl.ds(rr % 2, 1)],
                       sem_e.at[rr % 2])
    return 0

  lax.fori_loop(0, nrow + 1, _step, 0)
  for half in range(2):
    pltpu.make_async_copy(stgs[half], agg_sh.at[sidxs[half]],
                          sem_ss[half]).wait()

  plsc.subcore_barrier()

  for k5 in range(ZROWS // RB):
    start = pl.multiple_of(s * ZROWS + k5 * RB, 8)

    @pl.when(start + RB <= N_NODES)
    def _():
      pltpu.sync_copy(agg_sh.at[pl.ds(start, RB)], r0)
      pltpu.sync_copy(r0, apart.at[c, pl.ds(start, RB)])

  @pl.when(s == NS - 1)
  def _():
    pltpu.sync_copy(agg_sh.at[pl.ds(N_NODES - 16, 16)], r0.at[pl.ds(0, 16)])
    pltpu.sync_copy(r0.at[pl.ds(0, 16)], apart.at[c, pl.ds(N_NODES - 16, 16)])


def _agg_call(pk2, ew3, h):
  return pl.kernel(
      _agg_body,
      out_type=jax.ShapeDtypeStruct((NC, N_NODES, D), jnp.float32),
      mesh=_mesh(),
      scratch_types=(
          [
              pltpu.VMEM((NR_HI, RB), jnp.int32),
              pltpu.VMEM((2 * RB3, D), jnp.float32),
              pltpu.VMEM((RB3, D), jnp.float32),
              pltpu.VMEM((RB3, D), jnp.float32),
              pltpu.VMEM((2, RB), jnp.float32),
              pltpu.VMEM((RB3,), jnp.int32),
              pltpu.VMEM((RB3,), jnp.int32),
              pltpu.VMEM((RB3,), jnp.int32),
              pltpu.VMEM((RB3,), jnp.int32),
          ]
          + [pltpu.VMEM_SHARED((N_NODES, D), jnp.float32)]
          + [pltpu.SemaphoreType.DMA((2,)),
             pltpu.SemaphoreType.DMA((2,)),
             pltpu.SemaphoreType.DMA,
             pltpu.SemaphoreType.DMA]
      ),
  )(pk2, ew3, h)


RBLK = 1000


def _head_body(a2_ref, di_ref, w_ref, gng_ref, gnb_ref, gna_ref,
               w1_ref, b1_ref, w2_ref, b2_ref, out_ref,
               s1, s2, mx, mn):
  i = pl.program_id(0)
  a = (a2_ref[0] + a2_ref[1]) * di_ref[...]
  h = jnp.dot(a, w_ref[...], preferred_element_type=jnp.float32)
  h = jnp.where(h > 0, h, 0.01 * h)
  bs1 = jnp.sum(h, axis=0, keepdims=True)
  bs2 = jnp.sum(h * h, axis=0, keepdims=True)
  bmx = jnp.max(h, axis=0, keepdims=True)
  bmn = jnp.min(h, axis=0, keepdims=True)

  @pl.when(i == 0)
  def _():
    s1[...] = bs1
    s2[...] = bs2
    mx[...] = bmx
    mn[...] = bmn

  @pl.when(i > 0)
  def _():
    s1[...] += bs1
    s2[...] += bs2
    mx[...] = jnp.maximum(mx[...], bmx)
    mn[...] = jnp.minimum(mn[...], bmn)

  @pl.when(i == pl.num_programs(0) - 1)
  def _():
    nf = jnp.float32(N_NODES)
    mu = s1[...] / nf
    al = gna_ref[...]
    var = s2[...] / nf + mu * mu * (al * al - 2.0 * al)
    sig = lax.rsqrt(var + EPS)
    gam = gng_ref[...]
    bet = gnb_ref[...]
    gs = gam * sig
    g_mean = gs * (mu - al * mu) + bet
    hc_mx = mx[...] - al * mu
    hc_mn = mn[...] - al * mu
    g_max = jnp.where(gam >= 0, gs * hc_mx, gs * hc_mn) + bet
    g_min = jnp.where(gam >= 0, gs * hc_mn, gs * hc_mx) + bet
    g = jnp.concatenate([g_mean, g_max, g_min], axis=1)
    hid = jnp.maximum(
        jnp.dot(g, w1_ref[...], preferred_element_type=jnp.float32)
        + b1_ref[...], 0.0)
    out_ref[...] = (jnp.dot(hid, w2_ref[...], preferred_element_type=jnp.float32)
                    + b2_ref[...])


def _head_call(apart, dinvi2, W, gng, gnb, gna, W1, b1, W2, b2):
  nsteps = N_NODES // RBLK
  full = lambda shape: pl.BlockSpec(shape, lambda i: tuple(0 for _ in shape))
  return pl.pallas_call(
      _head_body,
      grid=(nsteps,),
      in_specs=[
          pl.BlockSpec((NC, RBLK, D), lambda i: (0, i, 0)),
          pl.BlockSpec((RBLK, 1), lambda i: (i, 0)),
          full((D, D)),
          full((1, D)),
          full((1, D)),
          full((1, D)),
          full((3 * D, 256)),
          full((1, 256)),
          full((256, OUT_D)),
          full((1, OUT_D)),
      ],
      out_specs=pl.BlockSpec((1, OUT_D), lambda i: (0, 0)),
      out_shape=jax.ShapeDtypeStruct((1, OUT_D), jnp.float32),
      scratch_shapes=[pltpu.VMEM((1, D), jnp.float32) for _ in range(4)],
  )(apart, dinvi2, W, gng, gnb, gna, W1, b1, W2, b2)


def kernel(x, edge_index, edge_weights, W, gn_gamma, gn_beta, gn_alpha,
           W1, b1, W2, b2):
  epad = jnp.pad(edge_index, ((0, 0), (0, N_PAD_EDGES)))
  edge3 = epad.reshape(2, NR_TOT, RB)
  pk2 = (epad[0] | (epad[1] << 16)).reshape(NR_TOT, RB)
  ew3 = jnp.pad(edge_weights, (0, N_PAD_EDGES)).reshape(NR_TOT, 1, RB)
  deg_part = _deg_call(edge3)
  h, dinvi2 = _prep_call(deg_part, x)
  apart = _agg_call(pk2, ew3, h)
  return _head_call(
      apart, dinvi2, W,
      gn_gamma.reshape(1, D), gn_beta.reshape(1, D), gn_alpha.reshape(1, D),
      W1, b1.reshape(1, -1), W2, b2.reshape(1, -1))


# speedup vs baseline: 10.8825x; 1.0388x over previous
import jax
import jax.numpy as jnp
from jax import lax
from jax.experimental import pallas as pl
from jax.experimental.pallas import tpu as pltpu
from jax.experimental.pallas import tpu_sc as plsc

N_NODES = 10000
N_EDGES = 320000
D = 128
OUT_D = 16
EPS = 1e-05

NC = 2
NS = 16
NW = NC * NS
RB = 128
NR_TOT = 2504
E_PAD = NR_TOT * RB
N_PAD_EDGES = E_PAD - N_EDGES

NR_HI = 80
NR_LO = 72
NR_SPLIT = 25

RB3 = 64

ZROWS = 640


def _mesh():
  return plsc.VectorSubcoreMesh(
      core_axis_name="c", subcore_axis_name="s", num_cores=NC, num_subcores=NS)


def _chunk(w):
  rbase = jnp.where(w < NR_SPLIT, NR_HI * w,
                    NR_HI * NR_SPLIT + NR_LO * (w - NR_SPLIT))
  nr = jnp.where(w < NR_SPLIT, NR_HI, NR_LO)
  return pl.multiple_of(rbase, 8), nr


def _stage_chunk(w, rbase, edge3, ew2, src_i, dst_i, ew_v):
  @pl.when(w < NR_SPLIT)
  def _():
    pltpu.sync_copy(edge3.at[0, pl.ds(rbase, NR_HI)], src_i)
    pltpu.sync_copy(edge3.at[1, pl.ds(rbase, NR_HI)], dst_i)
    if ew_v is not None:
      pltpu.sync_copy(ew2.at[pl.ds(rbase, NR_HI)], ew_v)

  @pl.when(w >= NR_SPLIT)
  def _():
    pltpu.sync_copy(edge3.at[0, pl.ds(rbase, NR_LO)], src_i.at[pl.ds(0, NR_LO)])
    pltpu.sync_copy(edge3.at[1, pl.ds(rbase, NR_LO)], dst_i.at[pl.ds(0, NR_LO)])
    if ew_v is not None:
      pltpu.sync_copy(ew2.at[pl.ds(rbase, NR_LO)], ew_v.at[pl.ds(0, NR_LO)])


def _deg_body(edge3, dpart, od_sh, id_sh, src_i, dst_i, ones_v, zbuf, sem_k):
  c = lax.axis_index("c")
  s = lax.axis_index("s")
  w = c * NS + s

  z16 = jnp.zeros((16,), jnp.float32)
  o16 = jnp.ones((16,), jnp.float32)

  def _zb(i, _):
    zbuf[pl.ds(i * 16, 16)] = z16
    return 0

  lax.fori_loop(0, N_NODES // 16, _zb, 0)
  for k in range(RB // 16):
    ones_v[pl.ds(k * 16, 16)] = o16

  @pl.when(s == 0)
  def _():
    pltpu.sync_copy(zbuf, od_sh)

  @pl.when(s == 1)
  def _():
    pltpu.sync_copy(zbuf, id_sh)

  rbase, nr = _chunk(w)
  _stage_chunk(w, rbase, edge3, None, src_i, dst_i, None)

  plsc.subcore_barrier()

  def _round(r, _):
    pltpu.async_copy(ones_v, od_sh.at[src_i.at[r]], sem_k, add=True)
    pltpu.async_copy(ones_v, id_sh.at[dst_i.at[r]], sem_k, add=True)

    @pl.when(r >= 2)
    def _():
      pltpu.make_async_copy(ones_v, od_sh.at[src_i.at[r - 2]], sem_k).wait()
      pltpu.make_async_copy(ones_v, id_sh.at[dst_i.at[r - 2]], sem_k).wait()

    return 0

  lax.fori_loop(0, nr, _round, 0)
  for k in (2, 1):
    pltpu.make_async_copy(ones_v, od_sh.at[src_i.at[nr - k]], sem_k).wait()
    pltpu.make_async_copy(ones_v, id_sh.at[dst_i.at[nr - k]], sem_k).wait()
  plsc.subcore_barrier()

  @pl.when(s == 0)
  def _():
    off = pl.multiple_of(2 * c * N_NODES, 8)
    pltpu.sync_copy(od_sh, zbuf)
    pltpu.sync_copy(zbuf, dpart.at[pl.ds(off, N_NODES)])

  @pl.when(s == 1)
  def _():
    off = pl.multiple_of((2 * c + 1) * N_NODES, 8)
    pltpu.sync_copy(id_sh, zbuf)
    pltpu.sync_copy(zbuf, dpart.at[pl.ds(off, N_NODES)])


def _deg_call(edge3):
  return pl.kernel(
      _deg_body,
      out_type=jax.ShapeDtypeStruct((4 * N_NODES,), jnp.float32),
      mesh=_mesh(),
      scratch_types=[
          pltpu.VMEM_SHARED((N_NODES,), jnp.float32),
          pltpu.VMEM_SHARED((N_NODES,), jnp.float32),
          pltpu.VMEM((NR_HI, RB), jnp.int32),
          pltpu.VMEM((NR_HI, RB), jnp.int32),
          pltpu.VMEM((RB,), jnp.float32),
          pltpu.VMEM((N_NODES,), jnp.float32),
          pltpu.SemaphoreType.DMA,
      ],
  )(edge3)


PBLK = 1000


def _prep_body(dp_ref, x_ref, h_ref, di_ref):
  i = pl.program_id(0)
  dp = dp_ref[...]
  rows0 = lax.broadcasted_iota(jnp.int32, (PBLK, 1), 0) == 0
  corr = jnp.where(rows0 & (i == 0), jnp.float32(N_PAD_EDGES), 0.0)
  od = jnp.maximum(dp[0] + dp[2] - corr, 1.0)
  idg = jnp.maximum(dp[1] + dp[3] - corr, 1.0)
  h_ref[...] = x_ref[...] * lax.rsqrt(od)
  di_ref[...] = lax.rsqrt(idg)


def _prep_call(deg_part, x):
  dp3 = deg_part.reshape(4, N_NODES, 1)
  return pl.pallas_call(
      _prep_body,
      grid=(N_NODES // PBLK,),
      in_specs=[
          pl.BlockSpec((4, PBLK, 1), lambda i: (0, i, 0)),
          pl.BlockSpec((PBLK, D), lambda i: (i, 0)),
      ],
      out_specs=[
          pl.BlockSpec((PBLK, D), lambda i: (i, 0)),
          pl.BlockSpec((PBLK, 1), lambda i: (i, 0)),
      ],
      out_shape=[
          jax.ShapeDtypeStruct((N_NODES, D), jnp.float32),
          jax.ShapeDtypeStruct((N_NODES, 1), jnp.float32),
      ],
  )(dp3, x)


def _agg_body(pk2, ew3, h, apart,
              pk_i, gbig, stg0, stg1, ew_roll, gidx0, gidx1, sidx0, sidx1,
              agg_sh, sem_g, sem_e, sem_s0, sem_s1):
  c = lax.axis_index("c")
  s = lax.axis_index("s")
  w = c * NS + s
  r0 = gbig.at[pl.ds(0, RB)]

  z16 = jnp.zeros((16,), jnp.float32)

  def _zr(e, _):
    for ct in range(D // 16):
      r0[e, pl.ds(ct * 16, 16)] = z16
    return 0

  lax.fori_loop(0, RB, _zr, 0)

  for k5 in range(ZROWS // RB):
    start = pl.multiple_of(s * ZROWS + k5 * RB, 8)

    @pl.when(start + RB <= N_NODES)
    def _():
      pltpu.sync_copy(r0, agg_sh.at[pl.ds(start, RB)])

  @pl.when(s == NS - 1)
  def _():
    pltpu.sync_copy(r0.at[pl.ds(0, 16)], agg_sh.at[pl.ds(N_NODES - 16, 16)])

  rbase, nrow = _chunk(w)

  @pl.when(w < NR_SPLIT)
  def _():
    pltpu.sync_copy(pk2.at[pl.ds(rbase, NR_HI)], pk_i)

  @pl.when(w >= NR_SPLIT)
  def _():
    pltpu.sync_copy(pk2.at[pl.ds(rbase, NR_LO)], pk_i.at[pl.ds(0, NR_LO)])

  plsc.subcore_barrier()

  gbufs = [gbig.at[pl.ds(0, RB3)], gbig.at[pl.ds(RB3, RB3)]]
  stgs = [stg0, stg1]
  gidxs = [gidx0, gidx1]
  sidxs = [sidx0, sidx1]
  sem_ss = [sem_s0, sem_s1]

  def _scale(gb, stg, row, half):
    par = row % 2
    for grp in range(RB3 // 16):
      sv16 = ew_roll[par, pl.ds(half * RB3 + grp * 16, 16)]
      for lane in range(16):
        e = grp * 16 + lane
        sp = sv16.at[jnp.full((16,), lane, jnp.int32)].get(
            mode="promise_in_bounds")
        for ct in range(D // 16):
          stg[e, pl.ds(ct * 16, 16)] = gb[e, pl.ds(ct * 16, 16)] * sp

  def _step(rr, _):
    row = rr - 1

    @pl.when(row >= 0)
    def _():
      pltpu.make_async_copy(ew3.at[rbase + row],
                            ew_roll.at[pl.ds(row % 2, 1)],
                            sem_e.at[row % 2]).wait()

    for half in range(2):
      gb = gbufs[half]
      stg = stgs[half]
      sidx = sidxs[half]
      sem_s = sem_ss[half]

      @pl.when(row >= 0)
      def _():
        pltpu.make_async_copy(h.at[gidxs[half]], gb, sem_g.at[half]).wait()

        @pl.when(row >= 1)
        def _():
          pltpu.make_async_copy(stg, agg_sh.at[sidx], sem_s).wait()

        _scale(gb, stg, row, half)
        for k in range(RB3 // 16):
          sidx[pl.ds(k * 16, 16)] = jnp.right_shift(
              pk_i[row, pl.ds(half * RB3 + k * 16, 16)], 16)
        pltpu.async_copy(stg, agg_sh.at[sidx], sem_s, add=True)

      @pl.when(rr < nrow)
      def _():
        for k in range(RB3 // 16):
          gidxs[half][pl.ds(k * 16, 16)] = (
              pk_i[rr, pl.ds(half * RB3 + k * 16, 16)] & 0xFFFF)
        pltpu.async_copy(h.at[gidxs[half]], gb, sem_g.at[half])

    @pl.when(rr < nrow)
    def _():
      pltpu.async_copy(ew3.at[rbase + rr], ew_roll.at[pl.ds(rr % 2, 1)],
                       sem_e.at[rr % 2])
    return 0

  lax.fori_loop(0, nrow + 1, _step, 0)
  for half in range(2):
    pltpu.make_async_copy(stgs[half], agg_sh.at[sidxs[half]],
                          sem_ss[half]).wait()

  plsc.subcore_barrier()

  for k5 in range(ZROWS // RB):
    start = pl.multiple_of(s * ZROWS + k5 * RB, 8)

    @pl.when(start + RB <= N_NODES)
    def _():
      pltpu.sync_copy(agg_sh.at[pl.ds(start, RB)], r0)
      pltpu.sync_copy(r0, apart.at[c, pl.ds(start, RB)])

  @pl.when(s == NS - 1)
  def _():
    pltpu.sync_copy(agg_sh.at[pl.ds(N_NODES - 16, 16)], r0.at[pl.ds(0, 16)])
    pltpu.sync_copy(r0.at[pl.ds(0, 16)], apart.at[c, pl.ds(N_NODES - 16, 16)])


def _agg_call(pk2, ew3, h):
  return pl.kernel(
      _agg_body,
      out_type=jax.ShapeDtypeStruct((NC, N_NODES, D), jnp.float32),
      mesh=_mesh(),
      scratch_types=(
          [
              pltpu.VMEM((NR_HI, RB), jnp.int32),
              pltpu.VMEM((2 * RB3, D), jnp.float32),
              pltpu.VMEM((RB3, D), jnp.float32),
              pltpu.VMEM((RB3, D), jnp.float32),
              pltpu.VMEM((2, RB), jnp.float32),
              pltpu.VMEM((RB3,), jnp.int32),
              pltpu.VMEM((RB3,), jnp.int32),
              pltpu.VMEM((RB3,), jnp.int32),
              pltpu.VMEM((RB3,), jnp.int32),
          ]
          + [pltpu.VMEM_SHARED((N_NODES, D), jnp.float32)]
          + [pltpu.SemaphoreType.DMA((2,)),
             pltpu.SemaphoreType.DMA((2,)),
             pltpu.SemaphoreType.DMA,
             pltpu.SemaphoreType.DMA]
      ),
  )(pk2, ew3, h)


RBLK = 1000


def _head_body(a2_ref, di_ref, w_ref, gng_ref, gnb_ref, gna_ref,
               w1_ref, b1_ref, w2_ref, b2_ref, out_ref,
               s1, s2, mx, mn):
  i = pl.program_id(0)
  a = (a2_ref[0] + a2_ref[1]) * di_ref[...]
  h = jnp.dot(a, w_ref[...], preferred_element_type=jnp.float32)
  h = jnp.where(h > 0, h, 0.01 * h)
  bs1 = jnp.sum(h, axis=0, keepdims=True)
  bs2 = jnp.sum(h * h, axis=0, keepdims=True)
  bmx = jnp.max(h, axis=0, keepdims=True)
  bmn = jnp.min(h, axis=0, keepdims=True)

  @pl.when(i == 0)
  def _():
    s1[...] = bs1
    s2[...] = bs2
    mx[...] = bmx
    mn[...] = bmn

  @pl.when(i > 0)
  def _():
    s1[...] += bs1
    s2[...] += bs2
    mx[...] = jnp.maximum(mx[...], bmx)
    mn[...] = jnp.minimum(mn[...], bmn)

  @pl.when(i == pl.num_programs(0) - 1)
  def _():
    nf = jnp.float32(N_NODES)
    mu = s1[...] / nf
    al = gna_ref[...]
    var = s2[...] / nf + mu * mu * (al * al - 2.0 * al)
    sig = lax.rsqrt(var + EPS)
    gam = gng_ref[...]
    bet = gnb_ref[...]
    gs = gam * sig
    g_mean = gs * (mu - al * mu) + bet
    hc_mx = mx[...] - al * mu
    hc_mn = mn[...] - al * mu
    g_max = jnp.where(gam >= 0, gs * hc_mx, gs * hc_mn) + bet
    g_min = jnp.where(gam >= 0, gs * hc_mn, gs * hc_mx) + bet
    g = jnp.concatenate([g_mean, g_max, g_min], axis=1)
    hid = jnp.maximum(
        jnp.dot(g, w1_ref[...], preferred_element_type=jnp.float32)
        + b1_ref[...], 0.0)
    out_ref[...] = (jnp.dot(hid, w2_ref[...], preferred_element_type=jnp.float32)
                    + b2_ref[...])


def _head_call(apart, dinvi2, W, gng, gnb, gna, W1, b1, W2, b2):
  nsteps = N_NODES // RBLK
  full = lambda shape: pl.BlockSpec(shape, lambda i: tuple(0 for _ in shape))
  return pl.pallas_call(
      _head_body,
      grid=(nsteps,),
      in_specs=[
          pl.BlockSpec((NC, RBLK, D), lambda i: (0, i, 0)),
          pl.BlockSpec((RBLK, 1), lambda i: (i, 0)),
          full((D, D)),
          full((1, D)),
          full((1, D)),
          full((1, D)),
          full((3 * D, 256)),
          full((1, 256)),
          full((256, OUT_D)),
          full((1, OUT_D)),
      ],
      out_specs=pl.BlockSpec((1, OUT_D), lambda i: (0, 0)),
      out_shape=jax.ShapeDtypeStruct((1, OUT_D), jnp.float32),
      scratch_shapes=[pltpu.VMEM((1, D), jnp.float32) for _ in range(4)],
  )(apart, dinvi2, W, gng, gnb, gna, W1, b1, W2, b2)


def kernel(x, edge_index, edge_weights, W, gn_gamma, gn_beta, gn_alpha,
           W1, b1, W2, b2):
  epad = jnp.pad(edge_index, ((0, 0), (0, N_PAD_EDGES)))
  edge3 = epad.reshape(2, NR_TOT, RB)
  pk2 = (epad[0] | (epad[1] << 16)).reshape(NR_TOT, RB)
  ew3 = jnp.pad(edge_weights, (0, N_PAD_EDGES)).reshape(NR_TOT, 1, RB)
  deg_part = _deg_call(edge3)
  h, dinvi2 = _prep_call(deg_part, x)
  apart = _agg_call(pk2, ew3, h)
  return _head_call(
      apart, dinvi2, W,
      gn_gamma.reshape(1, D), gn_beta.reshape(1, D), gn_alpha.reshape(1, D),
      W1, b1.reshape(1, -1), W2, b2.reshape(1, -1))
